# baseline (speedup 1.0000x reference)
import jax
import jax.numpy as jnp
from jax import lax
from jax.experimental import pallas as pl
from jax.experimental.pallas import tpu as pltpu
from jax.experimental.pallas import tpu_sc as plsc

N_NODES = 10000
C_FEAT = 128
N_EDGES = 320000

NC = 2
NS = 16
NW = NC * NS
EDGES_PER_TILE = N_EDGES // NW
CHUNK = 100
NCHUNK = EDGES_PER_TILE // CHUNK
GROUP = 5
NGROUP = NCHUNK // GROUP
PAD_N = 10240
STRIPE = PAD_N // NS
WB = 64
DEG_W = 8

_sc_mesh = plsc.VectorSubcoreMesh(core_axis_name="c", subcore_axis_name="s",
                                  num_cores=NC, num_subcores=NS)


def _deg_body(row4, ones_in, zer_in, degp, row_b, ones_v, tmp_v, deg_sp, ssem):
    c = lax.axis_index("c")
    s = lax.axis_index("s")
    wid = c * NS + s
    pltpu.sync_copy(ones_in, ones_v)
    pltpu.sync_copy(zer_in, tmp_v)
    for k in range(STRIPE // WB):
        pltpu.sync_copy(tmp_v, deg_sp.at[pl.ds(s * STRIPE + k * WB, WB)])
    plsc.subcore_barrier()

    def body(g, carry):
        pltpu.sync_copy(row4.at[wid, g], row_b)
        for k in range(GROUP):
            pltpu.async_copy(ones_v, deg_sp.at[row_b.at[k]], ssem, add=True)
        for k in range(GROUP):
            pltpu.make_async_copy(ones_v, deg_sp.at[row_b.at[k]], ssem).wait()
        return carry

    lax.fori_loop(0, NGROUP, body, 0)
    plsc.subcore_barrier()
    for k in range(STRIPE // WB):
        pltpu.sync_copy(deg_sp.at[pl.ds(s * STRIPE + k * WB, WB)], tmp_v)
        pltpu.sync_copy(tmp_v, degp.at[c, pl.ds(s * STRIPE + k * WB, WB)])


_deg_call = pl.kernel(
    _deg_body,
    out_type=jax.ShapeDtypeStruct((NC, PAD_N, C_FEAT), jnp.float32),
    mesh=_sc_mesh,
    scratch_types=[
        pltpu.VMEM((GROUP, CHUNK), jnp.int32),
        pltpu.VMEM((CHUNK, C_FEAT), jnp.float32),
        pltpu.VMEM((WB, C_FEAT), jnp.float32),
        pltpu.VMEM_SHARED((PAD_N, C_FEAT), jnp.float32),
        pltpu.SemaphoreType.DMA,
    ],
)


def _agg_body(z1, row4, col4, zer_in, part,
              row_b, col_b, gbuf0, gbuf1, tmp_v, acc_sp,
              gsem0, gsem1, ssem0, ssem1):
    c = lax.axis_index("c")
    s = lax.axis_index("s")
    wid = c * NS + s
    pltpu.sync_copy(zer_in, tmp_v)
    for k in range(STRIPE // WB):
        pltpu.sync_copy(tmp_v, acc_sp.at[pl.ds(s * STRIPE + k * WB, WB)])
    plsc.subcore_barrier()
    gb = (gbuf0, gbuf1)
    gs = (gsem0, gsem1)
    ss = (ssem0, ssem1)

    def body(g, carry):
        pltpu.sync_copy(row4.at[wid, g], row_b)
        pltpu.sync_copy(col4.at[wid, g], col_b)
        pltpu.async_copy(z1.at[col_b.at[0]], gbuf0, gsem0)
        for k in range(GROUP):
            cur = k % 2
            nxt = (k + 1) % 2
            pltpu.make_async_copy(z1.at[col_b.at[k]], gb[cur], gs[cur]).wait()
            if k >= 1:
                pltpu.make_async_copy(gb[nxt], acc_sp.at[row_b.at[k - 1]],
                                      ss[nxt]).wait()
            if k + 1 < GROUP:
                pltpu.async_copy(z1.at[col_b.at[k + 1]], gb[nxt], gs[nxt])
            pltpu.async_copy(gb[cur], acc_sp.at[row_b.at[k]], ss[cur], add=True)
        last = (GROUP - 1) % 2
        pltpu.make_async_copy(gb[last], acc_sp.at[row_b.at[GROUP - 1]],
                              ss[last]).wait()
        return carry

    lax.fori_loop(0, NGROUP, body, 0)
    plsc.subcore_barrier()
    for k in range(STRIPE // WB):
        pltpu.sync_copy(acc_sp.at[pl.ds(s * STRIPE + k * WB, WB)], tmp_v)
        pltpu.sync_copy(tmp_v, part.at[c, pl.ds(s * STRIPE + k * WB, WB)])


_agg_call = pl.kernel(
    _agg_body,
    out_type=jax.ShapeDtypeStruct((NC, PAD_N, C_FEAT), jnp.float32),
    mesh=_sc_mesh,
    scratch_types=[
        pltpu.VMEM((GROUP, CHUNK), jnp.int32),
        pltpu.VMEM((GROUP, CHUNK), jnp.int32),
        pltpu.VMEM((CHUNK, C_FEAT), jnp.float32),
        pltpu.VMEM((CHUNK, C_FEAT), jnp.float32),
        pltpu.VMEM((WB, C_FEAT), jnp.float32),
        pltpu.VMEM_SHARED((PAD_N, C_FEAT), jnp.float32),
        pltpu.SemaphoreType.DMA,
        pltpu.SemaphoreType.DMA,
        pltpu.SemaphoreType.DMA,
        pltpu.SemaphoreType.DMA,
    ],
)

_TC_BLK = 1000


def _tc1_body(x_ref, w_ref, deg_ref, z1_ref):
    d = lax.rsqrt(deg_ref[...] + 1.0)
    z = lax.dot_general(x_ref[...], w_ref[...], (((1,), (1,)), ((), ())),
                        preferred_element_type=jnp.float32)
    z1_ref[...] = z * d


def _tc2_body(z1_ref, part_ref, deg_ref, out_ref):
    d = lax.rsqrt(deg_ref[...] + 1.0)
    out_ref[...] = (z1_ref[...] + part_ref[0] + part_ref[1]) * d


def kernel(x, edge_index, W):
    row4 = edge_index[0].reshape(NW, NGROUP, GROUP, CHUNK)
    col4 = edge_index[1].reshape(NW, NGROUP, GROUP, CHUNK)
    ones128 = jnp.ones((CHUNK, C_FEAT), jnp.float32)
    zer128 = jnp.zeros((WB, C_FEAT), jnp.float32)

    degp = _deg_call(row4, ones128, zer128)
    deg = (degp[0, :N_NODES, 0] + degp[1, :N_NODES, 0])[:, None]

    z1 = pl.pallas_call(
        _tc1_body,
        grid=(N_NODES // _TC_BLK,),
        in_specs=[
            pl.BlockSpec((_TC_BLK, C_FEAT), lambda i: (i, 0)),
            pl.BlockSpec((C_FEAT, C_FEAT), lambda i: (0, 0)),
            pl.BlockSpec((_TC_BLK, 1), lambda i: (i, 0)),
        ],
        out_specs=pl.BlockSpec((_TC_BLK, C_FEAT), lambda i: (i, 0)),
        out_shape=jax.ShapeDtypeStruct((N_NODES, C_FEAT), jnp.float32),
    )(x, W, deg)

    part = _agg_call(z1, row4, col4, zer128)

    out = pl.pallas_call(
        _tc2_body,
        grid=(N_NODES // _TC_BLK,),
        in_specs=[
            pl.BlockSpec((_TC_BLK, C_FEAT), lambda i: (i, 0)),
            pl.BlockSpec((NC, _TC_BLK, C_FEAT), lambda i: (0, i, 0)),
            pl.BlockSpec((_TC_BLK, 1), lambda i: (i, 0)),
        ],
        out_specs=pl.BlockSpec((_TC_BLK, C_FEAT), lambda i: (i, 0)),
        out_shape=jax.ShapeDtypeStruct((N_NODES, C_FEAT), jnp.float32),
    )(z1, part, deg)
    return out


# speedup vs baseline: 1.0420x; 1.0420x over previous
import jax
import jax.numpy as jnp
from jax import lax
from jax.experimental import pallas as pl
from jax.experimental.pallas import tpu as pltpu
from jax.experimental.pallas import tpu_sc as plsc

N_NODES = 10000
C_FEAT = 128
N_EDGES = 320000

NC = 2
NS = 16
NW = NC * NS
EDGES_PER_TILE = N_EDGES // NW
CHUNK = 100
NCHUNK = EDGES_PER_TILE // CHUNK
GROUP = 5
NGROUP = NCHUNK // GROUP
AGROUP = 10
ANGROUP = NCHUNK // AGROUP
PAD_N = 10240
STRIPE = PAD_N // NS
WB = 64
DEG_W = 8

_sc_mesh = plsc.VectorSubcoreMesh(core_axis_name="c", subcore_axis_name="s",
                                  num_cores=NC, num_subcores=NS)


def _deg_body(row4, ones_in, zer_in, degp, row_b, ones_v, tmp_v, deg_sp, ssem):
    c = lax.axis_index("c")
    s = lax.axis_index("s")
    wid = c * NS + s
    pltpu.sync_copy(ones_in, ones_v)
    pltpu.sync_copy(zer_in, tmp_v)
    for k in range(STRIPE // WB):
        pltpu.sync_copy(tmp_v, deg_sp.at[pl.ds(s * STRIPE + k * WB, WB)])
    plsc.subcore_barrier()

    def body(g, carry):
        pltpu.sync_copy(row4.at[wid, g], row_b)
        for k in range(GROUP):
            pltpu.async_copy(ones_v, deg_sp.at[row_b.at[k]], ssem, add=True)
        for k in range(GROUP):
            pltpu.make_async_copy(ones_v, deg_sp.at[row_b.at[k]], ssem).wait()
        return carry

    lax.fori_loop(0, NGROUP, body, 0)
    plsc.subcore_barrier()
    for k in range(STRIPE // WB):
        pltpu.sync_copy(deg_sp.at[pl.ds(s * STRIPE + k * WB, WB)], tmp_v)
        pltpu.sync_copy(tmp_v, degp.at[c, pl.ds(s * STRIPE + k * WB, WB)])


_deg_call = pl.kernel(
    _deg_body,
    out_type=jax.ShapeDtypeStruct((NC, PAD_N, C_FEAT), jnp.float32),
    mesh=_sc_mesh,
    scratch_types=[
        pltpu.VMEM((GROUP, CHUNK), jnp.int32),
        pltpu.VMEM((CHUNK, C_FEAT), jnp.float32),
        pltpu.VMEM((WB, C_FEAT), jnp.float32),
        pltpu.VMEM_SHARED((PAD_N, C_FEAT), jnp.float32),
        pltpu.SemaphoreType.DMA,
    ],
)


def _agg_body(z1, row4, col4, zer_in, part,
              row_b, col_b, gbuf0, gbuf1, tmp_v, acc_sp,
              gsem0, gsem1, ssem0, ssem1):
    c = lax.axis_index("c")
    s = lax.axis_index("s")
    wid = c * NS + s
    pltpu.sync_copy(zer_in, tmp_v)
    for k in range(STRIPE // WB):
        pltpu.sync_copy(tmp_v, acc_sp.at[pl.ds(s * STRIPE + k * WB, WB)])
    plsc.subcore_barrier()
    gb = (gbuf0, gbuf1)
    gs = (gsem0, gsem1)
    ss = (ssem0, ssem1)

    def body(g, carry):
        pltpu.sync_copy(row4.at[wid, g], row_b)
        pltpu.sync_copy(col4.at[wid, g], col_b)
        pltpu.async_copy(z1.at[col_b.at[0]], gbuf0, gsem0)
        for k in range(AGROUP):
            cur = k % 2
            nxt = (k + 1) % 2
            pltpu.make_async_copy(z1.at[col_b.at[k]], gb[cur], gs[cur]).wait()
            if k + 1 < AGROUP:
                pltpu.async_copy(z1.at[col_b.at[k + 1]], gb[nxt], gs[nxt])
            pltpu.sync_copy(gb[cur], acc_sp.at[row_b.at[k]], add=True)
        return carry

    lax.fori_loop(0, ANGROUP, body, 0)
    plsc.subcore_barrier()
    for k in range(STRIPE // WB):
        pltpu.sync_copy(acc_sp.at[pl.ds(s * STRIPE + k * WB, WB)], tmp_v)
        pltpu.sync_copy(tmp_v, part.at[c, pl.ds(s * STRIPE + k * WB, WB)])


_agg_call = pl.kernel(
    _agg_body,
    out_type=jax.ShapeDtypeStruct((NC, PAD_N, C_FEAT), jnp.float32),
    mesh=_sc_mesh,
    scratch_types=[
        pltpu.VMEM((AGROUP, CHUNK), jnp.int32),
        pltpu.VMEM((AGROUP, CHUNK), jnp.int32),
        pltpu.VMEM((CHUNK, C_FEAT), jnp.float32),
        pltpu.VMEM((CHUNK, C_FEAT), jnp.float32),
        pltpu.VMEM((WB, C_FEAT), jnp.float32),
        pltpu.VMEM_SHARED((PAD_N, C_FEAT), jnp.float32),
        pltpu.SemaphoreType.DMA,
        pltpu.SemaphoreType.DMA,
        pltpu.SemaphoreType.DMA,
        pltpu.SemaphoreType.DMA,
    ],
)

_TC_BLK = 1000


def _tc1_body(x_ref, w_ref, deg_ref, z1_ref):
    d = lax.rsqrt(deg_ref[...] + 1.0)
    z = lax.dot_general(x_ref[...], w_ref[...], (((1,), (1,)), ((), ())),
                        preferred_element_type=jnp.float32)
    z1_ref[...] = z * d


def _tc2_body(z1_ref, part_ref, deg_ref, out_ref):
    d = lax.rsqrt(deg_ref[...] + 1.0)
    out_ref[...] = (z1_ref[...] + part_ref[0] + part_ref[1]) * d


def kernel(x, edge_index, W):
    row4 = edge_index[0].reshape(NW, NGROUP, GROUP, CHUNK)
    rowa = edge_index[0].reshape(NW, ANGROUP, AGROUP, CHUNK)
    cola = edge_index[1].reshape(NW, ANGROUP, AGROUP, CHUNK)
    ones128 = jnp.ones((CHUNK, C_FEAT), jnp.float32)
    zer128 = jnp.zeros((WB, C_FEAT), jnp.float32)

    degp = _deg_call(row4, ones128, zer128)
    deg = (degp[0, :N_NODES, 0] + degp[1, :N_NODES, 0])[:, None]

    z1 = pl.pallas_call(
        _tc1_body,
        grid=(N_NODES // _TC_BLK,),
        in_specs=[
            pl.BlockSpec((_TC_BLK, C_FEAT), lambda i: (i, 0)),
            pl.BlockSpec((C_FEAT, C_FEAT), lambda i: (0, 0)),
            pl.BlockSpec((_TC_BLK, 1), lambda i: (i, 0)),
        ],
        out_specs=pl.BlockSpec((_TC_BLK, C_FEAT), lambda i: (i, 0)),
        out_shape=jax.ShapeDtypeStruct((N_NODES, C_FEAT), jnp.float32),
    )(x, W, deg)

    part = _agg_call(z1, rowa, cola, zer128)

    out = pl.pallas_call(
        _tc2_body,
        grid=(N_NODES // _TC_BLK,),
        in_specs=[
            pl.BlockSpec((_TC_BLK, C_FEAT), lambda i: (i, 0)),
            pl.BlockSpec((NC, _TC_BLK, C_FEAT), lambda i: (0, i, 0)),
            pl.BlockSpec((_TC_BLK, 1), lambda i: (i, 0)),
        ],
        out_specs=pl.BlockSpec((_TC_BLK, C_FEAT), lambda i: (i, 0)),
        out_shape=jax.ShapeDtypeStruct((N_NODES, C_FEAT), jnp.float32),
    )(z1, part, deg)
    return out


# speedup vs baseline: 1.0580x; 1.0154x over previous
import jax
import jax.numpy as jnp
from jax import lax
from jax.experimental import pallas as pl
from jax.experimental.pallas import tpu as pltpu
from jax.experimental.pallas import tpu_sc as plsc

N_NODES = 10000
C_FEAT = 128
N_EDGES = 320000

NC = 2
NS = 16
NW = NC * NS
EDGES_PER_TILE = N_EDGES // NW
CHUNK = 100
NCHUNK = EDGES_PER_TILE // CHUNK
GROUP = 5
NGROUP = NCHUNK // GROUP
AGROUP = 10
ANGROUP = NCHUNK // AGROUP
PAD_N = 10240
STRIPE = PAD_N // NS
WB = 64
DEG_W = 8

_sc_mesh = plsc.VectorSubcoreMesh(core_axis_name="c", subcore_axis_name="s",
                                  num_cores=NC, num_subcores=NS)


def _deg_body(row4, ones_in, zer_in, degp, row_b, ones_v, tmp_v, deg_sp, ssem):
    c = lax.axis_index("c")
    s = lax.axis_index("s")
    wid = c * NS + s
    pltpu.sync_copy(ones_in, ones_v)
    pltpu.sync_copy(zer_in, tmp_v)
    for k in range(STRIPE // WB):
        pltpu.sync_copy(tmp_v, deg_sp.at[pl.ds(s * STRIPE + k * WB, WB)])
    plsc.subcore_barrier()

    def body(g, carry):
        pltpu.sync_copy(row4.at[wid, g], row_b)
        for k in range(GROUP):
            pltpu.async_copy(ones_v, deg_sp.at[row_b.at[k]], ssem, add=True)
        for k in range(GROUP):
            pltpu.make_async_copy(ones_v, deg_sp.at[row_b.at[k]], ssem).wait()
        return carry

    lax.fori_loop(0, NGROUP, body, 0)
    plsc.subcore_barrier()
    for k in range(STRIPE // WB):
        pltpu.sync_copy(deg_sp.at[pl.ds(s * STRIPE + k * WB, WB)], tmp_v)
        pltpu.sync_copy(tmp_v, degp.at[c, pl.ds(s * STRIPE + k * WB, WB)])


_deg_call = pl.kernel(
    _deg_body,
    out_type=jax.ShapeDtypeStruct((NC, PAD_N, C_FEAT), jnp.float32),
    mesh=_sc_mesh,
    scratch_types=[
        pltpu.VMEM((GROUP, CHUNK), jnp.int32),
        pltpu.VMEM((CHUNK, C_FEAT), jnp.float32),
        pltpu.VMEM((WB, C_FEAT), jnp.float32),
        pltpu.VMEM_SHARED((PAD_N, C_FEAT), jnp.float32),
        pltpu.SemaphoreType.DMA,
    ],
)


def _agg_body(z1, row4, col4, zer_in, part,
              row_b, col_b, gbuf0, gbuf1, tmp_v, acc_sp,
              gsem0, gsem1, ssem0, ssem1):
    c = lax.axis_index("c")
    s = lax.axis_index("s")
    wid = c * NS + s
    pltpu.sync_copy(zer_in, tmp_v)
    for k in range(STRIPE // WB):
        pltpu.sync_copy(tmp_v, acc_sp.at[pl.ds(s * STRIPE + k * WB, WB)])
    plsc.subcore_barrier()
    gb = (gbuf0, gbuf1)
    gs = (gsem0, gsem1)
    ss = (ssem0, ssem1)

    def body(g, carry):
        pltpu.sync_copy(row4.at[wid, g], row_b)
        pltpu.sync_copy(col4.at[wid, g], col_b)
        pltpu.async_copy(z1.at[col_b.at[0]], gbuf0, gsem0)
        for k in range(AGROUP):
            cur = k % 2
            nxt = (k + 1) % 2
            pltpu.make_async_copy(z1.at[col_b.at[k]], gb[cur], gs[cur]).wait()
            if k + 1 < AGROUP:
                pltpu.async_copy(z1.at[col_b.at[k + 1]], gb[nxt], gs[nxt])
            pltpu.sync_copy(gb[cur], acc_sp.at[row_b.at[k]], add=True)
        return carry

    lax.fori_loop(0, ANGROUP, body, 0)
    plsc.subcore_barrier()
    for k in range(STRIPE // WB):
        pltpu.sync_copy(acc_sp.at[pl.ds(s * STRIPE + k * WB, WB)], tmp_v)
        pltpu.sync_copy(tmp_v, part.at[c, pl.ds(s * STRIPE + k * WB, WB)])


_agg_call = pl.kernel(
    _agg_body,
    out_type=jax.ShapeDtypeStruct((NC, PAD_N, C_FEAT), jnp.float32),
    mesh=_sc_mesh,
    scratch_types=[
        pltpu.VMEM((AGROUP, CHUNK), jnp.int32),
        pltpu.VMEM((AGROUP, CHUNK), jnp.int32),
        pltpu.VMEM((CHUNK, C_FEAT), jnp.float32),
        pltpu.VMEM((CHUNK, C_FEAT), jnp.float32),
        pltpu.VMEM((WB, C_FEAT), jnp.float32),
        pltpu.VMEM_SHARED((PAD_N, C_FEAT), jnp.float32),
        pltpu.SemaphoreType.DMA,
        pltpu.SemaphoreType.DMA,
        pltpu.SemaphoreType.DMA,
        pltpu.SemaphoreType.DMA,
    ],
)

_TC_BLK = 1000


def _tc1_body(x_ref, w_ref, degp_ref, z1_ref):
    deg = degp_ref[0, :, 0:1] + degp_ref[1, :, 0:1]
    d = lax.rsqrt(deg + 1.0)
    z = lax.dot_general(x_ref[...], w_ref[...], (((1,), (1,)), ((), ())),
                        preferred_element_type=jnp.float32)
    z1_ref[...] = z * d


def _tc2_body(z1_ref, part_ref, degp_ref, out_ref):
    deg = degp_ref[0, :, 0:1] + degp_ref[1, :, 0:1]
    d = lax.rsqrt(deg + 1.0)
    out_ref[...] = (z1_ref[...] + part_ref[0] + part_ref[1]) * d


def kernel(x, edge_index, W):
    row4 = edge_index[0].reshape(NW, NGROUP, GROUP, CHUNK)
    rowa = edge_index[0].reshape(NW, ANGROUP, AGROUP, CHUNK)
    cola = edge_index[1].reshape(NW, ANGROUP, AGROUP, CHUNK)
    ones128 = jnp.ones((CHUNK, C_FEAT), jnp.float32)
    zer128 = jnp.zeros((WB, C_FEAT), jnp.float32)

    degp = _deg_call(row4, ones128, zer128)

    z1 = pl.pallas_call(
        _tc1_body,
        grid=(N_NODES // _TC_BLK,),
        in_specs=[
            pl.BlockSpec((_TC_BLK, C_FEAT), lambda i: (i, 0)),
            pl.BlockSpec((C_FEAT, C_FEAT), lambda i: (0, 0)),
            pl.BlockSpec((NC, _TC_BLK, C_FEAT), lambda i: (0, i, 0)),
        ],
        out_specs=pl.BlockSpec((_TC_BLK, C_FEAT), lambda i: (i, 0)),
        out_shape=jax.ShapeDtypeStruct((N_NODES, C_FEAT), jnp.float32),
    )(x, W, degp)

    part = _agg_call(z1, rowa, cola, zer128)

    out = pl.pallas_call(
        _tc2_body,
        grid=(N_NODES // _TC_BLK,),
        in_specs=[
            pl.BlockSpec((_TC_BLK, C_FEAT), lambda i: (i, 0)),
            pl.BlockSpec((NC, _TC_BLK, C_FEAT), lambda i: (0, i, 0)),
            pl.BlockSpec((NC, _TC_BLK, C_FEAT), lambda i: (0, i, 0)),
        ],
        out_specs=pl.BlockSpec((_TC_BLK, C_FEAT), lambda i: (i, 0)),
        out_shape=jax.ShapeDtypeStruct((N_NODES, C_FEAT), jnp.float32),
    )(z1, part, degp)
    return out


# speedup vs baseline: 1.0957x; 1.0356x over previous
import jax
import jax.numpy as jnp
from jax import lax
from jax.experimental import pallas as pl
from jax.experimental.pallas import tpu as pltpu
from jax.experimental.pallas import tpu_sc as plsc

N_NODES = 10000
C_FEAT = 128
N_EDGES = 320000

NC = 2
NS = 16
NW = NC * NS
EDGES_PER_TILE = N_EDGES // NW
CHUNK = 125
NCHUNK = EDGES_PER_TILE // CHUNK
GROUP = 5
NGROUP = NCHUNK // GROUP
AGROUP = 8
ANGROUP = NCHUNK // AGROUP
PAD_N = 10240
STRIPE = PAD_N // NS
WB = 32
DEG_W = 8

_sc_mesh = plsc.VectorSubcoreMesh(core_axis_name="c", subcore_axis_name="s",
                                  num_cores=NC, num_subcores=NS)


def _deg_body(row4, ones_in, zer_in, degp, row_b, ones_v, tmp_v, deg_sp, ssem):
    c = lax.axis_index("c")
    s = lax.axis_index("s")
    wid = c * NS + s
    pltpu.sync_copy(ones_in, ones_v)
    pltpu.sync_copy(zer_in, tmp_v)
    for k in range(STRIPE // WB):
        pltpu.sync_copy(tmp_v, deg_sp.at[pl.ds(s * STRIPE + k * WB, WB)])
    plsc.subcore_barrier()

    def body(g, carry):
        pltpu.sync_copy(row4.at[wid, g], row_b)
        for k in range(GROUP):
            pltpu.async_copy(ones_v, deg_sp.at[row_b.at[k]], ssem, add=True)
        for k in range(GROUP):
            pltpu.make_async_copy(ones_v, deg_sp.at[row_b.at[k]], ssem).wait()
        return carry

    lax.fori_loop(0, NGROUP, body, 0)
    plsc.subcore_barrier()
    for k in range(STRIPE // WB):
        pltpu.sync_copy(deg_sp.at[pl.ds(s * STRIPE + k * WB, WB)], tmp_v)
        pltpu.sync_copy(tmp_v, degp.at[c, pl.ds(s * STRIPE + k * WB, WB)])


_deg_call = pl.kernel(
    _deg_body,
    out_type=jax.ShapeDtypeStruct((NC, PAD_N, C_FEAT), jnp.float32),
    mesh=_sc_mesh,
    scratch_types=[
        pltpu.VMEM((GROUP, CHUNK), jnp.int32),
        pltpu.VMEM((CHUNK, C_FEAT), jnp.float32),
        pltpu.VMEM((WB, C_FEAT), jnp.float32),
        pltpu.VMEM_SHARED((PAD_N, C_FEAT), jnp.float32),
        pltpu.SemaphoreType.DMA,
    ],
)


def _agg_body(z1, row4, col4, zer_in, part,
              row_b, col_b, gbuf0, gbuf1, tmp_v, acc_sp,
              gsem0, gsem1, ssem0, ssem1):
    c = lax.axis_index("c")
    s = lax.axis_index("s")
    wid = c * NS + s
    pltpu.sync_copy(zer_in, tmp_v)
    for k in range(STRIPE // WB):
        pltpu.sync_copy(tmp_v, acc_sp.at[pl.ds(s * STRIPE + k * WB, WB)])
    plsc.subcore_barrier()
    gb = (gbuf0, gbuf1)
    gs = (gsem0, gsem1)
    ss = (ssem0, ssem1)

    def body(g, carry):
        pltpu.sync_copy(row4.at[wid, g], row_b)
        pltpu.sync_copy(col4.at[wid, g], col_b)
        pltpu.async_copy(z1.at[col_b.at[0]], gbuf0, gsem0)
        for k in range(AGROUP):
            cur = k % 2
            nxt = (k + 1) % 2
            pltpu.make_async_copy(z1.at[col_b.at[k]], gb[cur], gs[cur]).wait()
            if k + 1 < AGROUP:
                pltpu.async_copy(z1.at[col_b.at[k + 1]], gb[nxt], gs[nxt])
            pltpu.sync_copy(gb[cur], acc_sp.at[row_b.at[k]], add=True)
        return carry

    lax.fori_loop(0, ANGROUP, body, 0)
    plsc.subcore_barrier()
    for k in range(STRIPE // WB):
        pltpu.sync_copy(acc_sp.at[pl.ds(s * STRIPE + k * WB, WB)], tmp_v)
        pltpu.sync_copy(tmp_v, part.at[c, pl.ds(s * STRIPE + k * WB, WB)])


_agg_call = pl.kernel(
    _agg_body,
    out_type=jax.ShapeDtypeStruct((NC, PAD_N, C_FEAT), jnp.float32),
    mesh=_sc_mesh,
    scratch_types=[
        pltpu.VMEM((AGROUP, CHUNK), jnp.int32),
        pltpu.VMEM((AGROUP, CHUNK), jnp.int32),
        pltpu.VMEM((CHUNK, C_FEAT), jnp.float32),
        pltpu.VMEM((CHUNK, C_FEAT), jnp.float32),
        pltpu.VMEM((WB, C_FEAT), jnp.float32),
        pltpu.VMEM_SHARED((PAD_N, C_FEAT), jnp.float32),
        pltpu.SemaphoreType.DMA,
        pltpu.SemaphoreType.DMA,
        pltpu.SemaphoreType.DMA,
        pltpu.SemaphoreType.DMA,
    ],
)

_TC_BLK = 1000


def _tc1_body(x_ref, w_ref, degp_ref, z1_ref):
    deg = degp_ref[0, :, 0:1] + degp_ref[1, :, 0:1]
    d = lax.rsqrt(deg + 1.0)
    z = lax.dot_general(x_ref[...], w_ref[...], (((1,), (1,)), ((), ())),
                        preferred_element_type=jnp.float32)
    z1_ref[...] = z * d


def _tc2_body(z1_ref, part_ref, degp_ref, out_ref):
    deg = degp_ref[0, :, 0:1] + degp_ref[1, :, 0:1]
    d = lax.rsqrt(deg + 1.0)
    out_ref[...] = (z1_ref[...] + part_ref[0] + part_ref[1]) * d


def kernel(x, edge_index, W):
    row4 = edge_index[0].reshape(NW, NGROUP, GROUP, CHUNK)
    rowa = edge_index[0].reshape(NW, ANGROUP, AGROUP, CHUNK)
    cola = edge_index[1].reshape(NW, ANGROUP, AGROUP, CHUNK)
    ones128 = jnp.ones((CHUNK, C_FEAT), jnp.float32)
    zer128 = jnp.zeros((WB, C_FEAT), jnp.float32)

    degp = _deg_call(row4, ones128, zer128)

    z1 = pl.pallas_call(
        _tc1_body,
        grid=(N_NODES // _TC_BLK,),
        in_specs=[
            pl.BlockSpec((_TC_BLK, C_FEAT), lambda i: (i, 0)),
            pl.BlockSpec((C_FEAT, C_FEAT), lambda i: (0, 0)),
            pl.BlockSpec((NC, _TC_BLK, C_FEAT), lambda i: (0, i, 0)),
        ],
        out_specs=pl.BlockSpec((_TC_BLK, C_FEAT), lambda i: (i, 0)),
        out_shape=jax.ShapeDtypeStruct((N_NODES, C_FEAT), jnp.float32),
    )(x, W, degp)

    part = _agg_call(z1, rowa, cola, zer128)

    out = pl.pallas_call(
        _tc2_body,
        grid=(N_NODES // _TC_BLK,),
        in_specs=[
            pl.BlockSpec((_TC_BLK, C_FEAT), lambda i: (i, 0)),
            pl.BlockSpec((NC, _TC_BLK, C_FEAT), lambda i: (0, i, 0)),
            pl.BlockSpec((NC, _TC_BLK, C_FEAT), lambda i: (0, i, 0)),
        ],
        out_specs=pl.BlockSpec((_TC_BLK, C_FEAT), lambda i: (i, 0)),
        out_shape=jax.ShapeDtypeStruct((N_NODES, C_FEAT), jnp.float32),
    )(z1, part, degp)
    return out
